# baseline (speedup 1.0000x reference)
import functools

import jax
import jax.numpy as jnp
from jax import lax
from jax.experimental import pallas as pl
from jax.experimental.pallas import tpu as pltpu
from jax.experimental.pallas import tpu_sc as plsc

N = 10000
E = 160000
RANGE = 320
ACCR = 328
MACRO = 8000
NM = E // MACRO
SCAP = 24512
FLUSH = 16384
CAPT = E + 768
SB = 1024
GCH = 64
NPAD = 10240
RB = 1024
GRID = 10

_mesh = plsc.VectorSubcoreMesh(core_axis_name="c", subcore_axis_name="s")
_sc_params = pltpu.CompilerParams(needs_layout_passes=False)


def _al(x):
    return pl.multiple_of(x, 8)


@functools.partial(
    pl.kernel,
    out_type=[
        jax.ShapeDtypeStruct((NPAD,), jnp.float32),
        jax.ShapeDtypeStruct((32 * CAPT,), jnp.int32),
        jax.ShapeDtypeStruct((32 * CAPT,), jnp.int32),
        jax.ShapeDtypeStruct((512,), jnp.int32),
    ],
    mesh=_mesh,
    compiler_params=_sc_params,
    scratch_types=[
        pltpu.VMEM((MACRO,), jnp.int32),
        pltpu.VMEM((MACRO,), jnp.int32),
        pltpu.VMEM((MACRO,), jnp.int32),
        pltpu.VMEM((MACRO,), jnp.int32),
        pltpu.VMEM((SCAP,), jnp.int32),
        pltpu.VMEM((SCAP,), jnp.int32),
        pltpu.VMEM((ACCR * 16,), jnp.float32),
        pltpu.VMEM((RANGE,), jnp.float32),
        pltpu.VMEM((16,), jnp.int32),
        pltpu.SemaphoreType.DMA,
        pltpu.SemaphoreType.DMA,
    ],
)
def _pre_kernel(src_h, dst_h, deg_out, csrc_out, cdst_out, cnt_out,
                rsA, rdA, rsB, rdB, sts, std, hist, degv, cntv, semA, semB):
    c = lax.axis_index("c")
    s = lax.axis_index("s")
    w = c * 16 + s
    base = w * RANGE
    tile0 = w * CAPT
    lane = lax.iota(jnp.int32, 16)
    ones = jnp.ones((16,), jnp.float32)

    bufs = [(rsA, rdA, semA), (rsB, rdB, semB)]

    def fire(m):
        rs, rd, sem = bufs[m % 2]
        h1 = pltpu.async_copy(src_h.at[pl.ds(m * MACRO, MACRO)], rs, sem)
        h2 = pltpu.async_copy(dst_h.at[pl.ds(m * MACRO, MACRO)], rd, sem)
        return h1, h2

    handles = [None] * NM
    handles[0] = fire(0)

    def zh(i, carry):
        hist[pl.ds(i * 16, 16)] = jnp.zeros((16,), jnp.float32)
        return carry
    lax.fori_loop(0, ACCR, zh, 0)

    off = jnp.int32(0)
    hbase = jnp.int32(0)
    for m in range(NM):
        h1, h2 = handles[m]
        h1.wait()
        h2.wait()
        if m + 1 < NM:
            handles[m + 1] = fire(m + 1)
        rs, rd, _ = bufs[m % 2]

        def vec(j, off2, rs=rs, rd=rd):
            d = rd[pl.ds(j * 16, 16)]
            sv = rs[pl.ds(j * 16, 16)]
            l = d - base
            ok = (l >= 0) & (l < RANGE)
            hidx = jnp.where(ok, l, ACCR - 1) * 16 + lane
            hv = plsc.load_gather(hist, [hidx])
            plsc.store_scatter(hist, [hidx], hv + ones)
            plsc.store_compressed(std.at[pl.ds(off2, 16)], l, mask=ok)
            plsc.store_compressed(sts.at[pl.ds(off2, 16)], sv, mask=ok)
            return off2 + jnp.sum(jnp.where(ok, 1, 0))
        off = lax.fori_loop(0, MACRO // 16, vec, off)

        pred = off >= FLUSH

        @pl.when(pred)
        def _(off=off, hbase=hbase):
            tb = tile0 + hbase

            def fb(b, carry):
                pltpu.sync_copy(sts.at[pl.ds(_al(b * 1024), 1024)],
                                csrc_out.at[pl.ds(_al(tb + b * 1024), 1024)])
                pltpu.sync_copy(std.at[pl.ds(_al(b * 1024), 1024)],
                                cdst_out.at[pl.ds(_al(tb + b * 1024), 1024)])
                return carry
            lax.fori_loop(0, FLUSH // 1024, fb, 0)

            nmv = (off - FLUSH + 15) // 16

            def mv(i, carry):
                sts[pl.ds(i * 16, 16)] = sts[pl.ds(FLUSH + i * 16, 16)]
                std[pl.ds(i * 16, 16)] = std[pl.ds(FLUSH + i * 16, 16)]
                return carry
            lax.fori_loop(0, nmv, mv, 0)

        off = jnp.where(pred, off - FLUSH, off)
        hbase = jnp.where(pred, hbase + FLUSH, hbase)

    offp = ((off + 63) // 64) * 64
    p0 = (off // 16) * 16
    for k in range(5):
        idx = p0 + k * 16 + lane
        mask = (idx >= off) & (idx < offp)
        plsc.store_scatter(std, [idx], jnp.full((16,), RANGE, jnp.int32),
                           mask=mask)
        plsc.store_scatter(sts, [idx], jnp.zeros((16,), jnp.int32),
                           mask=mask)

    tb = tile0 + hbase
    nb = offp // 1024

    def f1(b, carry):
        pltpu.sync_copy(sts.at[pl.ds(_al(b * 1024), 1024)],
                        csrc_out.at[pl.ds(_al(tb + b * 1024), 1024)])
        pltpu.sync_copy(std.at[pl.ds(_al(b * 1024), 1024)],
                        cdst_out.at[pl.ds(_al(tb + b * 1024), 1024)])
        return carry
    lax.fori_loop(0, nb, f1, 0)

    r0 = nb * 1024
    rem = (offp - r0) // 64

    def f2(b, carry):
        pltpu.sync_copy(sts.at[pl.ds(_al(r0 + b * 64), 64)],
                        csrc_out.at[pl.ds(_al(tb + r0 + b * 64), 64)])
        pltpu.sync_copy(std.at[pl.ds(_al(r0 + b * 64), 64)],
                        cdst_out.at[pl.ds(_al(tb + r0 + b * 64), 64)])
        return carry
    lax.fori_loop(0, rem, f2, 0)

    ntr = (hbase + offp) // GCH
    cntv[pl.ds(0, 16)] = jnp.full((16,), ntr, jnp.int32)
    pltpu.sync_copy(cntv, cnt_out.at[pl.ds(w * 16, 16)])

    def red(g, carry):
        tot = jnp.zeros((16,), jnp.float32)
        for l in range(16):
            v = plsc.load_gather(hist, [(g * 16 + lane) * 16 + l])
            tot = tot + v
        degv[pl.ds(g * 16, 16)] = tot
        return carry
    lax.fori_loop(0, RANGE // 16, red, 0)

    pltpu.sync_copy(degv, deg_out.at[pl.ds(base, RANGE)])


@functools.partial(
    pl.kernel,
    out_type=jax.ShapeDtypeStruct((N, 256), jnp.float32),
    mesh=_mesh,
    compiler_params=_sc_params,
    scratch_types=[
        pltpu.VMEM((SB,), jnp.int32),
        pltpu.VMEM((SB,), jnp.int32),
        pltpu.VMEM((SB,), jnp.int32),
        pltpu.VMEM((SB,), jnp.int32),
        pltpu.VMEM((GCH, 256), jnp.float32),
        pltpu.VMEM((GCH, 256), jnp.float32),
        pltpu.VMEM((ACCR, 256), jnp.float32),
        pltpu.VMEM((16,), jnp.int32),
        pltpu.SemaphoreType.DMA,
        pltpu.SemaphoreType.DMA,
        pltpu.SemaphoreType.DMA,
        pltpu.SemaphoreType.DMA,
    ],
)
def _agg_kernel(hn, csrc, cdst, cnts, out, iA, lA, iB, lB, rowsA, rowsB,
                acc, cntv, semIA, semIB, semA, semB):
    c = lax.axis_index("c")
    s = lax.axis_index("s")
    w = c * 16 + s
    base = w * RANGE
    tile0 = w * CAPT
    lane = lax.iota(jnp.int32, 16)

    def za(i, carry):
        acc[i // 16, pl.ds((i % 16) * 16, 16)] = jnp.zeros((16,), jnp.float32)
        return carry
    lax.fori_loop(0, ACCR * 16, za, 0)

    pltpu.sync_copy(cnts.at[pl.ds(w * 16, 16)], cntv)
    ntrip = jnp.max(cntv[pl.ds(0, 16)])
    nsb = (ntrip + 15) // 16

    def fire_idx(sb, idxr, ldr, sem):
        pltpu.async_copy(csrc.at[pl.ds(_al(tile0 + sb * SB), SB)], idxr, sem)
        pltpu.async_copy(cdst.at[pl.ds(_al(tile0 + sb * SB), SB)], ldr, sem)

    def wait_idx(sb, idxr, ldr, sem):
        pltpu.make_async_copy(csrc.at[pl.ds(_al(tile0 + sb * SB), SB)], idxr,
                              sem).wait()
        pltpu.make_async_copy(cdst.at[pl.ds(_al(tile0 + sb * SB), SB)], ldr,
                              sem).wait()

    def accum(ldr, t, rowsr):
        def grp(j, carry2):
            lvec = ldr[pl.ds(t * GCH + j * 16, 16)]
            for i in range(16):
                ld = jnp.max(jnp.where(lane == i, lvec, 0))
                for cc in range(16):
                    plsc.addupdate(acc.at[ld, pl.ds(cc * 16, 16)],
                                   rowsr[j * 16 + i, pl.ds(cc * 16, 16)])
            return carry2
        lax.fori_loop(0, GCH // 16, grp, 0)

    def process(idxr, ldr, sb):
        nt = jnp.minimum(16, ntrip - sb * 16)

        def fire_rows(t, rowsr, sem):
            for sl in range(4):
                pltpu.async_copy(
                    hn.at[idxr.at[pl.ds(t * GCH + sl * 16, 16)]],
                    rowsr.at[pl.ds(sl * 16, 16)], sem)

        def wait_rows(t, rowsr, sem):
            for sl in range(4):
                pltpu.make_async_copy(
                    hn.at[idxr.at[pl.ds(t * GCH + sl * 16, 16)]],
                    rowsr.at[pl.ds(sl * 16, 16)], sem).wait()

        @pl.when(nt > 0)
        def _():
            fire_rows(jnp.int32(0), rowsA, semA)

        def trip(t, carry):
            even = (t % 2) == 0

            @pl.when(even)
            def _():
                wait_rows(t, rowsA, semA)

                @pl.when(t + 1 < nt)
                def _():
                    fire_rows(t + 1, rowsB, semB)
                accum(ldr, t, rowsA)

            @pl.when(jnp.logical_not(even))
            def _():
                wait_rows(t, rowsB, semB)

                @pl.when(t + 1 < nt)
                def _():
                    fire_rows(t + 1, rowsA, semA)
                accum(ldr, t, rowsB)
            return carry
        lax.fori_loop(0, nt, trip, 0)

    @pl.when(nsb > 0)
    def _():
        fire_idx(jnp.int32(0), iA, lA, semIA)

    def sbody(sb, carry):
        even = (sb % 2) == 0

        @pl.when(even)
        def _():
            wait_idx(sb, iA, lA, semIA)

            @pl.when(sb + 1 < nsb)
            def _():
                fire_idx(sb + 1, iB, lB, semIB)
            process(iA, lA, sb)

        @pl.when(jnp.logical_not(even))
        def _():
            wait_idx(sb, iB, lB, semIB)

            @pl.when(sb + 1 < nsb)
            def _():
                fire_idx(sb + 1, iA, lA, semIA)
            process(iB, lB, sb)
        return carry
    lax.fori_loop(0, nsb, sbody, 0)

    @pl.when(w < 31)
    def _():
        pltpu.sync_copy(acc.at[pl.ds(0, RANGE)], out.at[pl.ds(base, RANGE)])

    @pl.when(w == 31)
    def _():
        pltpu.sync_copy(acc.at[pl.ds(0, 80)], out.at[pl.ds(base, 80)])


def _mm1_body(x_ref, w_ref, deg_ref, out_ref):
    dinv = lax.rsqrt(deg_ref[...] + 1.0)
    h = jnp.dot(x_ref[...], w_ref[...], preferred_element_type=jnp.float32)
    out_ref[...] = h * dinv[:, None]


def _mid_body(s_ref, hn_ref, deg_ref, w_ref, b_ref, out_ref):
    dinv = lax.rsqrt(deg_ref[...] + 1.0)
    h1 = jnp.maximum(dinv[:, None] * (s_ref[...] + hn_ref[...])
                     + b_ref[...][None, :], 0.0)
    h2 = jnp.dot(h1, w_ref[...], preferred_element_type=jnp.float32)
    out_ref[...] = h2 * dinv[:, None]


def _head_body(s_ref, hn_ref, deg_ref, b2_ref, wd1_ref, bd1_ref, wd2_ref,
               bd2_ref, out_ref):
    dinv = lax.rsqrt(deg_ref[...] + 1.0)
    h2 = dinv[:, None] * (s_ref[...] + hn_ref[...]) + b2_ref[...][None, :]
    h3 = jnp.maximum(
        jnp.dot(h2, wd1_ref[...], preferred_element_type=jnp.float32)
        + bd1_ref[...][None, :], 0.0)
    out_ref[...] = (jnp.dot(h3, wd2_ref[...],
                            preferred_element_type=jnp.float32)
                    + bd2_ref[...][None, :])


def _row_spec(cols):
    return pl.BlockSpec((RB, cols), lambda i: (i, 0))


def _full_spec(shape):
    nd = len(shape)
    return pl.BlockSpec(shape, lambda i: (0,) * nd)


def _mm1(x, W1, deg):
    return pl.pallas_call(
        _mm1_body,
        grid=(GRID,),
        in_specs=[_row_spec(256), _full_spec((256, 256)),
                  pl.BlockSpec((RB,), lambda i: (i,))],
        out_specs=_row_spec(256),
        out_shape=jax.ShapeDtypeStruct((N, 256), jnp.float32),
    )(x, W1, deg)


def _mid(s1, hn1, deg, W2, b1):
    return pl.pallas_call(
        _mid_body,
        grid=(GRID,),
        in_specs=[_row_spec(256), _row_spec(256),
                  pl.BlockSpec((RB,), lambda i: (i,)),
                  _full_spec((256, 256)), _full_spec((256,))],
        out_specs=_row_spec(256),
        out_shape=jax.ShapeDtypeStruct((N, 256), jnp.float32),
    )(s1, hn1, deg, W2, b1)


def _head(s2, hn2, deg, b2, Wd1, bd1, Wd2, bd2):
    return pl.pallas_call(
        _head_body,
        grid=(GRID,),
        in_specs=[_row_spec(256), _row_spec(256),
                  pl.BlockSpec((RB,), lambda i: (i,)),
                  _full_spec((256,)), _full_spec((256, 512)),
                  _full_spec((512,)), _full_spec((512, 16)),
                  _full_spec((16,))],
        out_specs=_row_spec(16),
        out_shape=jax.ShapeDtypeStruct((N, 16), jnp.float32),
    )(s2, hn2, deg, b2, Wd1, bd1, Wd2, bd2)


def kernel(x, edge_index, W1, b1, W2, b2, Wd1, bd1, Wd2, bd2):
    deg, csrc, cdst, cnts = _pre_kernel(edge_index[0], edge_index[1])
    hn1 = _mm1(x, W1, deg)
    s1 = _agg_kernel(hn1, csrc, cdst, cnts)
    hn2 = _mid(s1, hn1, deg, W2, b1)
    s2 = _agg_kernel(hn2, csrc, cdst, cnts)
    out = _head(s2, hn2, deg, b2, Wd1, bd1, Wd2, bd2)
    return out


# speedup vs baseline: 1.2627x; 1.2627x over previous
import functools

import jax
import jax.numpy as jnp
from jax import lax
from jax.experimental import pallas as pl
from jax.experimental.pallas import tpu as pltpu
from jax.experimental.pallas import tpu_sc as plsc

N = 10000
E = 160000
RANGE = 320
ACCR = 328
MACRO = 8000
NM = E // MACRO
SCAP = 24512
FLUSH = 16384
CAPT = E + 768
SB = 1024
GCH = 64
NPAD = 10240
RB = 1024
GRID = 10

_mesh = plsc.VectorSubcoreMesh(core_axis_name="c", subcore_axis_name="s")
_sc_params = pltpu.CompilerParams(needs_layout_passes=False)


def _al(x):
    return pl.multiple_of(x, 8)


@functools.partial(
    pl.kernel,
    out_type=[
        jax.ShapeDtypeStruct((NPAD,), jnp.float32),
        jax.ShapeDtypeStruct((32 * CAPT,), jnp.int32),
        jax.ShapeDtypeStruct((32 * CAPT,), jnp.int32),
        jax.ShapeDtypeStruct((512,), jnp.int32),
    ],
    mesh=_mesh,
    compiler_params=_sc_params,
    scratch_types=[
        pltpu.VMEM((MACRO,), jnp.int32),
        pltpu.VMEM((MACRO,), jnp.int32),
        pltpu.VMEM((MACRO,), jnp.int32),
        pltpu.VMEM((MACRO,), jnp.int32),
        pltpu.VMEM((SCAP,), jnp.int32),
        pltpu.VMEM((SCAP,), jnp.int32),
        pltpu.VMEM((ACCR * 16,), jnp.float32),
        pltpu.VMEM((RANGE,), jnp.float32),
        pltpu.VMEM((16,), jnp.int32),
        pltpu.SemaphoreType.DMA,
        pltpu.SemaphoreType.DMA,
    ],
)
def _pre_kernel(src_h, dst_h, deg_out, csrc_out, cdst_out, cnt_out,
                rsA, rdA, rsB, rdB, sts, std, hist, degv, cntv, semA, semB):
    c = lax.axis_index("c")
    s = lax.axis_index("s")
    w = c * 16 + s
    base = w * RANGE
    tile0 = w * CAPT
    lane = lax.iota(jnp.int32, 16)
    ones = jnp.ones((16,), jnp.float32)

    bufs = [(rsA, rdA, semA), (rsB, rdB, semB)]

    def fire(m):
        rs, rd, sem = bufs[m % 2]
        h1 = pltpu.async_copy(src_h.at[pl.ds(m * MACRO, MACRO)], rs, sem)
        h2 = pltpu.async_copy(dst_h.at[pl.ds(m * MACRO, MACRO)], rd, sem)
        return h1, h2

    handles = [None] * NM
    handles[0] = fire(0)

    def zh(i, carry):
        hist[pl.ds(i * 16, 16)] = jnp.zeros((16,), jnp.float32)
        return carry
    lax.fori_loop(0, ACCR, zh, 0)

    off = jnp.int32(0)
    hbase = jnp.int32(0)
    for m in range(NM):
        h1, h2 = handles[m]
        h1.wait()
        h2.wait()
        if m + 1 < NM:
            handles[m + 1] = fire(m + 1)
        rs, rd, _ = bufs[m % 2]

        def vec(j, off2, rs=rs, rd=rd):
            d = rd[pl.ds(j * 16, 16)]
            sv = rs[pl.ds(j * 16, 16)]
            l = d - base
            ok = (l >= 0) & (l < RANGE)
            hidx = jnp.where(ok, l, ACCR - 1) * 16 + lane
            hv = plsc.load_gather(hist, [hidx])
            plsc.store_scatter(hist, [hidx], hv + ones)
            plsc.store_compressed(std.at[pl.ds(off2, 16)], l, mask=ok)
            plsc.store_compressed(sts.at[pl.ds(off2, 16)], sv, mask=ok)
            return off2 + jnp.sum(jnp.where(ok, 1, 0))
        off = lax.fori_loop(0, MACRO // 16, vec, off)

        pred = off >= FLUSH

        @pl.when(pred)
        def _(off=off, hbase=hbase):
            tb = tile0 + hbase

            def fb(b, carry):
                pltpu.sync_copy(sts.at[pl.ds(_al(b * 1024), 1024)],
                                csrc_out.at[pl.ds(_al(tb + b * 1024), 1024)])
                pltpu.sync_copy(std.at[pl.ds(_al(b * 1024), 1024)],
                                cdst_out.at[pl.ds(_al(tb + b * 1024), 1024)])
                return carry
            lax.fori_loop(0, FLUSH // 1024, fb, 0)

            nmv = (off - FLUSH + 15) // 16

            def mv(i, carry):
                sts[pl.ds(i * 16, 16)] = sts[pl.ds(FLUSH + i * 16, 16)]
                std[pl.ds(i * 16, 16)] = std[pl.ds(FLUSH + i * 16, 16)]
                return carry
            lax.fori_loop(0, nmv, mv, 0)

        off = jnp.where(pred, off - FLUSH, off)
        hbase = jnp.where(pred, hbase + FLUSH, hbase)

    offp = ((off + 63) // 64) * 64
    p0 = (off // 16) * 16
    for k in range(5):
        idx = p0 + k * 16 + lane
        mask = (idx >= off) & (idx < offp)
        plsc.store_scatter(std, [idx], jnp.full((16,), RANGE, jnp.int32),
                           mask=mask)
        plsc.store_scatter(sts, [idx], jnp.zeros((16,), jnp.int32),
                           mask=mask)

    tb = tile0 + hbase
    nb = offp // 1024

    def f1(b, carry):
        pltpu.sync_copy(sts.at[pl.ds(_al(b * 1024), 1024)],
                        csrc_out.at[pl.ds(_al(tb + b * 1024), 1024)])
        pltpu.sync_copy(std.at[pl.ds(_al(b * 1024), 1024)],
                        cdst_out.at[pl.ds(_al(tb + b * 1024), 1024)])
        return carry
    lax.fori_loop(0, nb, f1, 0)

    r0 = nb * 1024
    rem = (offp - r0) // 64

    def f2(b, carry):
        pltpu.sync_copy(sts.at[pl.ds(_al(r0 + b * 64), 64)],
                        csrc_out.at[pl.ds(_al(tb + r0 + b * 64), 64)])
        pltpu.sync_copy(std.at[pl.ds(_al(r0 + b * 64), 64)],
                        cdst_out.at[pl.ds(_al(tb + r0 + b * 64), 64)])
        return carry
    lax.fori_loop(0, rem, f2, 0)

    ntr = (hbase + offp) // GCH
    cntv[pl.ds(0, 16)] = jnp.full((16,), ntr, jnp.int32)
    pltpu.sync_copy(cntv, cnt_out.at[pl.ds(w * 16, 16)])

    def red(g, carry):
        tot = jnp.zeros((16,), jnp.float32)
        for l in range(16):
            v = plsc.load_gather(hist, [(g * 16 + lane) * 16 + l])
            tot = tot + v
        degv[pl.ds(g * 16, 16)] = tot
        return carry
    lax.fori_loop(0, RANGE // 16, red, 0)

    pltpu.sync_copy(degv, deg_out.at[pl.ds(base, RANGE)])


@functools.partial(
    pl.kernel,
    out_type=jax.ShapeDtypeStruct((N, 256), jnp.float32),
    mesh=_mesh,
    compiler_params=_sc_params,
    scratch_types=[
        pltpu.VMEM((SB,), jnp.int32),
        pltpu.VMEM((SB,), jnp.int32),
        pltpu.VMEM((SB,), jnp.int32),
        pltpu.VMEM((SB,), jnp.int32),
        pltpu.VMEM((GCH, 128), jnp.int32),
        pltpu.VMEM((GCH, 128), jnp.int32),
        pltpu.VMEM((ACCR, 256), jnp.float32),
        pltpu.VMEM((16,), jnp.int32),
        pltpu.SemaphoreType.DMA,
        pltpu.SemaphoreType.DMA,
        pltpu.SemaphoreType.DMA,
        pltpu.SemaphoreType.DMA,
    ],
)
def _agg_kernel(hn, csrc, cdst, cnts, out, iA, lA, iB, lB, rowsA, rowsB,
                acc, cntv, semIA, semIB, semA, semB):
    c = lax.axis_index("c")
    s = lax.axis_index("s")
    w = c * 16 + s
    base = w * RANGE
    tile0 = w * CAPT
    lane = lax.iota(jnp.int32, 16)

    def za(i, carry):
        acc[i // 16, pl.ds((i % 16) * 16, 16)] = jnp.zeros((16,), jnp.float32)
        return carry
    lax.fori_loop(0, ACCR * 16, za, 0)

    pltpu.sync_copy(cnts.at[pl.ds(w * 16, 16)], cntv)
    ntrip = jnp.max(cntv[pl.ds(0, 16)])
    nsb = (ntrip + 15) // 16

    def fire_idx(sb, idxr, ldr, sem):
        pltpu.async_copy(csrc.at[pl.ds(_al(tile0 + sb * SB), SB)], idxr, sem)
        pltpu.async_copy(cdst.at[pl.ds(_al(tile0 + sb * SB), SB)], ldr, sem)

    def wait_idx(sb, idxr, ldr, sem):
        pltpu.make_async_copy(csrc.at[pl.ds(_al(tile0 + sb * SB), SB)], idxr,
                              sem).wait()
        pltpu.make_async_copy(cdst.at[pl.ds(_al(tile0 + sb * SB), SB)], ldr,
                              sem).wait()

    def accum(ldr, t, rowsr):
        def grp(j, carry2):
            lvec = ldr[pl.ds(t * GCH + j * 16, 16)]
            for i in range(16):
                ld = jnp.max(jnp.where(lane == i, lvec, 0))
                for g in range(8):
                    v = rowsr[j * 16 + i, pl.ds(g * 16, 16)]
                    lo = plsc.bitcast(v << 16, jnp.float32)
                    hi = plsc.bitcast(v & jnp.int32(-65536), jnp.float32)
                    plsc.addupdate(acc.at[ld, pl.ds(g * 16, 16)], lo)
                    plsc.addupdate(acc.at[ld, pl.ds(128 + g * 16, 16)], hi)
            return carry2
        lax.fori_loop(0, GCH // 16, grp, 0)

    def process(idxr, ldr, sb):
        nt = jnp.minimum(16, ntrip - sb * 16)

        def fire_rows(t, rowsr, sem):
            pltpu.async_copy(hn.at[idxr.at[pl.ds(t * GCH, GCH)]], rowsr, sem)

        def wait_rows(t, rowsr, sem):
            pltpu.make_async_copy(hn.at[idxr.at[pl.ds(t * GCH, GCH)]], rowsr,
                                  sem).wait()

        @pl.when(nt > 0)
        def _():
            fire_rows(jnp.int32(0), rowsA, semA)

        def trip(t, carry):
            even = (t % 2) == 0

            @pl.when(even)
            def _():
                wait_rows(t, rowsA, semA)

                @pl.when(t + 1 < nt)
                def _():
                    fire_rows(t + 1, rowsB, semB)
                accum(ldr, t, rowsA)

            @pl.when(jnp.logical_not(even))
            def _():
                wait_rows(t, rowsB, semB)

                @pl.when(t + 1 < nt)
                def _():
                    fire_rows(t + 1, rowsA, semA)
                accum(ldr, t, rowsB)
            return carry
        lax.fori_loop(0, nt, trip, 0)

    @pl.when(nsb > 0)
    def _():
        fire_idx(jnp.int32(0), iA, lA, semIA)

    def sbody(sb, carry):
        even = (sb % 2) == 0

        @pl.when(even)
        def _():
            wait_idx(sb, iA, lA, semIA)

            @pl.when(sb + 1 < nsb)
            def _():
                fire_idx(sb + 1, iB, lB, semIB)
            process(iA, lA, sb)

        @pl.when(jnp.logical_not(even))
        def _():
            wait_idx(sb, iB, lB, semIB)

            @pl.when(sb + 1 < nsb)
            def _():
                fire_idx(sb + 1, iA, lA, semIA)
            process(iB, lB, sb)
        return carry
    lax.fori_loop(0, nsb, sbody, 0)

    @pl.when(w < 31)
    def _():
        pltpu.sync_copy(acc.at[pl.ds(0, RANGE)], out.at[pl.ds(base, RANGE)])

    @pl.when(w == 31)
    def _():
        pltpu.sync_copy(acc.at[pl.ds(0, 80)], out.at[pl.ds(base, 80)])


def _pack_bf16(hs):
    b = lax.bitcast_convert_type(hs, jnp.int32)
    bf = (b + jnp.int32(0x7FFF) + ((b >> 16) & 1)) >> 16
    return (bf[:, 128:] << 16) | (bf[:, :128] & jnp.int32(0xFFFF))


def _mm1_body(x_ref, w_ref, deg_ref, out_ref, outb_ref):
    dinv = lax.rsqrt(deg_ref[...] + 1.0)
    h = jnp.dot(x_ref[...], w_ref[...], preferred_element_type=jnp.float32)
    hs = h * dinv[:, None]
    out_ref[...] = hs
    outb_ref[...] = _pack_bf16(hs)


def _mid_body(s_ref, hn_ref, deg_ref, w_ref, b_ref, out_ref, outb_ref):
    dinv = lax.rsqrt(deg_ref[...] + 1.0)
    h1 = jnp.maximum(dinv[:, None] * (s_ref[...] + hn_ref[...])
                     + b_ref[...][None, :], 0.0)
    h2 = jnp.dot(h1, w_ref[...], preferred_element_type=jnp.float32)
    h2s = h2 * dinv[:, None]
    out_ref[...] = h2s
    outb_ref[...] = _pack_bf16(h2s)


def _head_body(s_ref, hn_ref, deg_ref, b2_ref, wd1_ref, bd1_ref, wd2_ref,
               bd2_ref, out_ref):
    dinv = lax.rsqrt(deg_ref[...] + 1.0)
    h2 = dinv[:, None] * (s_ref[...] + hn_ref[...]) + b2_ref[...][None, :]
    h3 = jnp.maximum(
        jnp.dot(h2, wd1_ref[...], preferred_element_type=jnp.float32)
        + bd1_ref[...][None, :], 0.0)
    out_ref[...] = (jnp.dot(h3, wd2_ref[...],
                            preferred_element_type=jnp.float32)
                    + bd2_ref[...][None, :])


def _row_spec(cols):
    return pl.BlockSpec((RB, cols), lambda i: (i, 0))


def _full_spec(shape):
    nd = len(shape)
    return pl.BlockSpec(shape, lambda i: (0,) * nd)


def _mm1(x, W1, deg):
    return pl.pallas_call(
        _mm1_body,
        grid=(GRID,),
        in_specs=[_row_spec(256), _full_spec((256, 256)),
                  pl.BlockSpec((RB,), lambda i: (i,))],
        out_specs=[_row_spec(256), _row_spec(128)],
        out_shape=[jax.ShapeDtypeStruct((N, 256), jnp.float32),
                   jax.ShapeDtypeStruct((N, 128), jnp.int32)],
    )(x, W1, deg)


def _mid(s1, hn1, deg, W2, b1):
    return pl.pallas_call(
        _mid_body,
        grid=(GRID,),
        in_specs=[_row_spec(256), _row_spec(256),
                  pl.BlockSpec((RB,), lambda i: (i,)),
                  _full_spec((256, 256)), _full_spec((256,))],
        out_specs=[_row_spec(256), _row_spec(128)],
        out_shape=[jax.ShapeDtypeStruct((N, 256), jnp.float32),
                   jax.ShapeDtypeStruct((N, 128), jnp.int32)],
    )(s1, hn1, deg, W2, b1)


def _head(s2, hn2, deg, b2, Wd1, bd1, Wd2, bd2):
    return pl.pallas_call(
        _head_body,
        grid=(GRID,),
        in_specs=[_row_spec(256), _row_spec(256),
                  pl.BlockSpec((RB,), lambda i: (i,)),
                  _full_spec((256,)), _full_spec((256, 512)),
                  _full_spec((512,)), _full_spec((512, 16)),
                  _full_spec((16,))],
        out_specs=_row_spec(16),
        out_shape=jax.ShapeDtypeStruct((N, 16), jnp.float32),
    )(s2, hn2, deg, b2, Wd1, bd1, Wd2, bd2)


def kernel(x, edge_index, W1, b1, W2, b2, Wd1, bd1, Wd2, bd2):
    deg, csrc, cdst, cnts = _pre_kernel(edge_index[0], edge_index[1])
    hn1, hn1b = _mm1(x, W1, deg)
    s1 = _agg_kernel(hn1b, csrc, cdst, cnts)
    hn2, hn2b = _mid(s1, hn1, deg, W2, b1)
    s2 = _agg_kernel(hn2b, csrc, cdst, cnts)
    out = _head(s2, hn2, deg, b2, Wd1, bd1, Wd2, bd2)
    return out


# speedup vs baseline: 1.3294x; 1.0528x over previous
import functools

import jax
import jax.numpy as jnp
from jax import lax
from jax.experimental import pallas as pl
from jax.experimental.pallas import tpu as pltpu
from jax.experimental.pallas import tpu_sc as plsc

N = 10000
E = 160000
RANGE = 320
ACCR = 328
MACRO = 8000
NM = E // MACRO
SCAP = 24512
FLUSH = 16384
CAPT = E + 768
SB = 1024
GCH = 64
NPAD = 10240
RB = 1024
GRID = 10

_mesh = plsc.VectorSubcoreMesh(core_axis_name="c", subcore_axis_name="s")
_sc_params = pltpu.CompilerParams(needs_layout_passes=False)


def _al(x):
    return pl.multiple_of(x, 8)


@functools.partial(
    pl.kernel,
    out_type=[
        jax.ShapeDtypeStruct((NPAD,), jnp.float32),
        jax.ShapeDtypeStruct((32 * CAPT,), jnp.int32),
        jax.ShapeDtypeStruct((32 * CAPT,), jnp.int32),
        jax.ShapeDtypeStruct((512,), jnp.int32),
    ],
    mesh=_mesh,
    compiler_params=_sc_params,
    scratch_types=[
        pltpu.VMEM((MACRO,), jnp.int32),
        pltpu.VMEM((MACRO,), jnp.int32),
        pltpu.VMEM((MACRO,), jnp.int32),
        pltpu.VMEM((MACRO,), jnp.int32),
        pltpu.VMEM((SCAP,), jnp.int32),
        pltpu.VMEM((SCAP,), jnp.int32),
        pltpu.VMEM((ACCR * 16,), jnp.float32),
        pltpu.VMEM((RANGE,), jnp.float32),
        pltpu.VMEM((16,), jnp.int32),
        pltpu.SemaphoreType.DMA,
        pltpu.SemaphoreType.DMA,
    ],
)
def _pre_kernel(src_h, dst_h, deg_out, csrc_out, cdst_out, cnt_out,
                rsA, rdA, rsB, rdB, sts, std, hist, degv, cntv, semA, semB):
    c = lax.axis_index("c")
    s = lax.axis_index("s")
    w = c * 16 + s
    base = w * RANGE
    tile0 = w * CAPT
    lane = lax.iota(jnp.int32, 16)
    ones = jnp.ones((16,), jnp.float32)

    bufs = [(rsA, rdA, semA), (rsB, rdB, semB)]

    def fire(m):
        rs, rd, sem = bufs[m % 2]
        h1 = pltpu.async_copy(src_h.at[pl.ds(m * MACRO, MACRO)], rs, sem)
        h2 = pltpu.async_copy(dst_h.at[pl.ds(m * MACRO, MACRO)], rd, sem)
        return h1, h2

    handles = [None] * NM
    handles[0] = fire(0)

    def zh(i, carry):
        hist[pl.ds(i * 16, 16)] = jnp.zeros((16,), jnp.float32)
        return carry
    lax.fori_loop(0, ACCR, zh, 0)

    off = jnp.int32(0)
    hbase = jnp.int32(0)
    for m in range(NM):
        h1, h2 = handles[m]
        h1.wait()
        h2.wait()
        if m + 1 < NM:
            handles[m + 1] = fire(m + 1)
        rs, rd, _ = bufs[m % 2]

        def vec(j, off2, rs=rs, rd=rd):
            d = rd[pl.ds(j * 16, 16)]
            sv = rs[pl.ds(j * 16, 16)]
            l = d - base
            ok = (l >= 0) & (l < RANGE)
            plsc.store_compressed(std.at[pl.ds(off2, 16)], l, mask=ok)
            plsc.store_compressed(sts.at[pl.ds(off2, 16)], sv, mask=ok)
            return off2 + jnp.sum(jnp.where(ok, 1, 0))
        off0 = off
        off = lax.fori_loop(0, MACRO // 16, vec, off)

        def hvec(i, carry):
            gidx = i * 16 + lane
            msk = (gidx >= off0) & (gidx < off)
            l = std[pl.ds(i * 16, 16)]
            hidx = jnp.where(msk, l * 16 + lane, ACCR * 16 - 16 + lane)
            hv = plsc.load_gather(hist, [hidx])
            plsc.store_scatter(hist, [hidx],
                               hv + jnp.where(msk, 1.0, 0.0))
            return carry
        lax.fori_loop(off0 // 16, (off + 15) // 16, hvec, 0)

        pred = off >= FLUSH

        @pl.when(pred)
        def _(off=off, hbase=hbase):
            tb = tile0 + hbase

            def fb(b, carry):
                pltpu.sync_copy(sts.at[pl.ds(_al(b * 1024), 1024)],
                                csrc_out.at[pl.ds(_al(tb + b * 1024), 1024)])
                pltpu.sync_copy(std.at[pl.ds(_al(b * 1024), 1024)],
                                cdst_out.at[pl.ds(_al(tb + b * 1024), 1024)])
                return carry
            lax.fori_loop(0, FLUSH // 1024, fb, 0)

            nmv = (off - FLUSH + 15) // 16

            def mv(i, carry):
                sts[pl.ds(i * 16, 16)] = sts[pl.ds(FLUSH + i * 16, 16)]
                std[pl.ds(i * 16, 16)] = std[pl.ds(FLUSH + i * 16, 16)]
                return carry
            lax.fori_loop(0, nmv, mv, 0)

        off = jnp.where(pred, off - FLUSH, off)
        hbase = jnp.where(pred, hbase + FLUSH, hbase)

    offp = ((off + 63) // 64) * 64
    p0 = (off // 16) * 16
    for k in range(5):
        idx = p0 + k * 16 + lane
        mask = (idx >= off) & (idx < offp)
        plsc.store_scatter(std, [idx], jnp.full((16,), RANGE, jnp.int32),
                           mask=mask)
        plsc.store_scatter(sts, [idx], jnp.zeros((16,), jnp.int32),
                           mask=mask)

    tb = tile0 + hbase
    nb = offp // 1024

    def f1(b, carry):
        pltpu.sync_copy(sts.at[pl.ds(_al(b * 1024), 1024)],
                        csrc_out.at[pl.ds(_al(tb + b * 1024), 1024)])
        pltpu.sync_copy(std.at[pl.ds(_al(b * 1024), 1024)],
                        cdst_out.at[pl.ds(_al(tb + b * 1024), 1024)])
        return carry
    lax.fori_loop(0, nb, f1, 0)

    r0 = nb * 1024
    rem = (offp - r0) // 64

    def f2(b, carry):
        pltpu.sync_copy(sts.at[pl.ds(_al(r0 + b * 64), 64)],
                        csrc_out.at[pl.ds(_al(tb + r0 + b * 64), 64)])
        pltpu.sync_copy(std.at[pl.ds(_al(r0 + b * 64), 64)],
                        cdst_out.at[pl.ds(_al(tb + r0 + b * 64), 64)])
        return carry
    lax.fori_loop(0, rem, f2, 0)

    ntr = (hbase + offp) // GCH
    cntv[pl.ds(0, 16)] = jnp.full((16,), ntr, jnp.int32)
    pltpu.sync_copy(cntv, cnt_out.at[pl.ds(w * 16, 16)])

    def red(g, carry):
        tot = jnp.zeros((16,), jnp.float32)
        for l in range(16):
            v = plsc.load_gather(hist, [(g * 16 + lane) * 16 + l])
            tot = tot + v
        degv[pl.ds(g * 16, 16)] = tot
        return carry
    lax.fori_loop(0, RANGE // 16, red, 0)

    pltpu.sync_copy(degv, deg_out.at[pl.ds(base, RANGE)])


@functools.partial(
    pl.kernel,
    out_type=jax.ShapeDtypeStruct((N, 256), jnp.float32),
    mesh=_mesh,
    compiler_params=_sc_params,
    scratch_types=[
        pltpu.VMEM((SB,), jnp.int32),
        pltpu.VMEM((SB,), jnp.int32),
        pltpu.VMEM((SB,), jnp.int32),
        pltpu.VMEM((SB,), jnp.int32),
        pltpu.VMEM((GCH, 128), jnp.int32),
        pltpu.VMEM((GCH, 128), jnp.int32),
        pltpu.VMEM((ACCR, 256), jnp.float32),
        pltpu.VMEM((16,), jnp.int32),
        pltpu.SemaphoreType.DMA,
        pltpu.SemaphoreType.DMA,
        pltpu.SemaphoreType.DMA,
        pltpu.SemaphoreType.DMA,
    ],
)
def _agg_kernel(hn, csrc, cdst, cnts, out, iA, lA, iB, lB, rowsA, rowsB,
                acc, cntv, semIA, semIB, semA, semB):
    c = lax.axis_index("c")
    s = lax.axis_index("s")
    w = c * 16 + s
    base = w * RANGE
    tile0 = w * CAPT
    lane = lax.iota(jnp.int32, 16)

    def za(i, carry):
        acc[i // 16, pl.ds((i % 16) * 16, 16)] = jnp.zeros((16,), jnp.float32)
        return carry
    lax.fori_loop(0, ACCR * 16, za, 0)

    pltpu.sync_copy(cnts.at[pl.ds(w * 16, 16)], cntv)
    ntrip = jnp.max(cntv[pl.ds(0, 16)])
    nsb = (ntrip + 15) // 16

    def fire_idx(sb, idxr, ldr, sem):
        pltpu.async_copy(csrc.at[pl.ds(_al(tile0 + sb * SB), SB)], idxr, sem)
        pltpu.async_copy(cdst.at[pl.ds(_al(tile0 + sb * SB), SB)], ldr, sem)

    def wait_idx(sb, idxr, ldr, sem):
        pltpu.make_async_copy(csrc.at[pl.ds(_al(tile0 + sb * SB), SB)], idxr,
                              sem).wait()
        pltpu.make_async_copy(cdst.at[pl.ds(_al(tile0 + sb * SB), SB)], ldr,
                              sem).wait()

    def accum(ldr, t, rowsr):
        def grp(j, carry2):
            lvec = ldr[pl.ds(t * GCH + j * 16, 16)]
            for i in range(16):
                ld = jnp.max(jnp.where(lane == i, lvec, 0))
                for g in range(8):
                    v = rowsr[j * 16 + i, pl.ds(g * 16, 16)]
                    lo = plsc.bitcast(v << 16, jnp.float32)
                    hi = plsc.bitcast(v & jnp.int32(-65536), jnp.float32)
                    plsc.addupdate(acc.at[ld, pl.ds(g * 16, 16)], lo)
                    plsc.addupdate(acc.at[ld, pl.ds(128 + g * 16, 16)], hi)
            return carry2
        lax.fori_loop(0, GCH // 16, grp, 0)

    def process(idxr, ldr, sb):
        nt = jnp.minimum(16, ntrip - sb * 16)

        def fire_rows(t, rowsr, sem):
            pltpu.async_copy(hn.at[idxr.at[pl.ds(t * GCH, GCH)]], rowsr, sem)

        def wait_rows(t, rowsr, sem):
            pltpu.make_async_copy(hn.at[idxr.at[pl.ds(t * GCH, GCH)]], rowsr,
                                  sem).wait()

        @pl.when(nt > 0)
        def _():
            fire_rows(jnp.int32(0), rowsA, semA)

        def trip(t, carry):
            even = (t % 2) == 0

            @pl.when(even)
            def _():
                wait_rows(t, rowsA, semA)

                @pl.when(t + 1 < nt)
                def _():
                    fire_rows(t + 1, rowsB, semB)
                accum(ldr, t, rowsA)

            @pl.when(jnp.logical_not(even))
            def _():
                wait_rows(t, rowsB, semB)

                @pl.when(t + 1 < nt)
                def _():
                    fire_rows(t + 1, rowsA, semA)
                accum(ldr, t, rowsB)
            return carry
        lax.fori_loop(0, nt, trip, 0)

    @pl.when(nsb > 0)
    def _():
        fire_idx(jnp.int32(0), iA, lA, semIA)

    def sbody(sb, carry):
        even = (sb % 2) == 0

        @pl.when(even)
        def _():
            wait_idx(sb, iA, lA, semIA)

            @pl.when(sb + 1 < nsb)
            def _():
                fire_idx(sb + 1, iB, lB, semIB)
            process(iA, lA, sb)

        @pl.when(jnp.logical_not(even))
        def _():
            wait_idx(sb, iB, lB, semIB)

            @pl.when(sb + 1 < nsb)
            def _():
                fire_idx(sb + 1, iA, lA, semIA)
            process(iB, lB, sb)
        return carry
    lax.fori_loop(0, nsb, sbody, 0)

    @pl.when(w < 31)
    def _():
        pltpu.sync_copy(acc.at[pl.ds(0, RANGE)], out.at[pl.ds(base, RANGE)])

    @pl.when(w == 31)
    def _():
        pltpu.sync_copy(acc.at[pl.ds(0, 80)], out.at[pl.ds(base, 80)])


def _pack_bf16(hs):
    b = lax.bitcast_convert_type(hs, jnp.int32)
    bf = (b + jnp.int32(0x7FFF) + ((b >> 16) & 1)) >> 16
    return (bf[:, 128:] << 16) | (bf[:, :128] & jnp.int32(0xFFFF))


def _mm1_body(x_ref, w_ref, deg_ref, out_ref, outb_ref):
    dinv = lax.rsqrt(deg_ref[...] + 1.0)
    h = jnp.dot(x_ref[...], w_ref[...], preferred_element_type=jnp.float32)
    hs = h * dinv[:, None]
    out_ref[...] = hs
    outb_ref[...] = _pack_bf16(hs)


def _mid_body(s_ref, hn_ref, deg_ref, w_ref, b_ref, out_ref, outb_ref):
    dinv = lax.rsqrt(deg_ref[...] + 1.0)
    h1 = jnp.maximum(dinv[:, None] * (s_ref[...] + hn_ref[...])
                     + b_ref[...][None, :], 0.0)
    h2 = jnp.dot(h1, w_ref[...], preferred_element_type=jnp.float32)
    h2s = h2 * dinv[:, None]
    out_ref[...] = h2s
    outb_ref[...] = _pack_bf16(h2s)


def _head_body(s_ref, hn_ref, deg_ref, b2_ref, wd1_ref, bd1_ref, wd2_ref,
               bd2_ref, out_ref):
    dinv = lax.rsqrt(deg_ref[...] + 1.0)
    h2 = dinv[:, None] * (s_ref[...] + hn_ref[...]) + b2_ref[...][None, :]
    h3 = jnp.maximum(
        jnp.dot(h2, wd1_ref[...], preferred_element_type=jnp.float32)
        + bd1_ref[...][None, :], 0.0)
    out_ref[...] = (jnp.dot(h3, wd2_ref[...],
                            preferred_element_type=jnp.float32)
                    + bd2_ref[...][None, :])


def _row_spec(cols):
    return pl.BlockSpec((RB, cols), lambda i: (i, 0))


def _full_spec(shape):
    nd = len(shape)
    return pl.BlockSpec(shape, lambda i: (0,) * nd)


def _mm1(x, W1, deg):
    return pl.pallas_call(
        _mm1_body,
        grid=(GRID,),
        in_specs=[_row_spec(256), _full_spec((256, 256)),
                  pl.BlockSpec((RB,), lambda i: (i,))],
        out_specs=[_row_spec(256), _row_spec(128)],
        out_shape=[jax.ShapeDtypeStruct((N, 256), jnp.float32),
                   jax.ShapeDtypeStruct((N, 128), jnp.int32)],
    )(x, W1, deg)


def _mid(s1, hn1, deg, W2, b1):
    return pl.pallas_call(
        _mid_body,
        grid=(GRID,),
        in_specs=[_row_spec(256), _row_spec(256),
                  pl.BlockSpec((RB,), lambda i: (i,)),
                  _full_spec((256, 256)), _full_spec((256,))],
        out_specs=[_row_spec(256), _row_spec(128)],
        out_shape=[jax.ShapeDtypeStruct((N, 256), jnp.float32),
                   jax.ShapeDtypeStruct((N, 128), jnp.int32)],
    )(s1, hn1, deg, W2, b1)


def _head(s2, hn2, deg, b2, Wd1, bd1, Wd2, bd2):
    return pl.pallas_call(
        _head_body,
        grid=(GRID,),
        in_specs=[_row_spec(256), _row_spec(256),
                  pl.BlockSpec((RB,), lambda i: (i,)),
                  _full_spec((256,)), _full_spec((256, 512)),
                  _full_spec((512,)), _full_spec((512, 16)),
                  _full_spec((16,))],
        out_specs=_row_spec(16),
        out_shape=jax.ShapeDtypeStruct((N, 16), jnp.float32),
    )(s2, hn2, deg, b2, Wd1, bd1, Wd2, bd2)


def kernel(x, edge_index, W1, b1, W2, b2, Wd1, bd1, Wd2, bd2):
    deg, csrc, cdst, cnts = _pre_kernel(edge_index[0], edge_index[1])
    hn1, hn1b = _mm1(x, W1, deg)
    s1 = _agg_kernel(hn1b, csrc, cdst, cnts)
    hn2, hn2b = _mid(s1, hn1, deg, W2, b1)
    s2 = _agg_kernel(hn2b, csrc, cdst, cnts)
    out = _head(s2, hn2, deg, b2, Wd1, bd1, Wd2, bd2)
    return out
